# baseline (speedup 1.0000x reference)
import functools

import jax
import jax.numpy as jnp
from jax import lax
from jax.experimental import pallas as pl
from jax.experimental.pallas import tpu as pltpu
from jax.experimental.pallas import tpu_sc as plsc

_NUM_CORES = 2
_NUM_SUBCORES = 16
_NW = _NUM_CORES * _NUM_SUBCORES
_L = 16


def _make_kernel(N, T, C, Q, G):
    assert N % (_NW * G) == 0 and C % _L == 0
    e_per_w = N // _NW
    ng = e_per_w // G
    in_w = T * C
    out_w = Q * C
    qb_n = Q // _L
    ch_n = C // _L

    mesh = plsc.VectorSubcoreMesh(core_axis_name="c", subcore_axis_name="s")

    @functools.partial(
        pl.kernel,
        out_type=jax.ShapeDtypeStruct((N * out_w,), jnp.float32),
        mesh=mesh,
        scratch_types=[
            pltpu.VMEM((T,), jnp.float32),
            pltpu.VMEM((Q,), jnp.float32),
            pltpu.VMEM((Q,), jnp.int32),
            pltpu.VMEM((Q,), jnp.int32),
            pltpu.VMEM((Q,), jnp.float32),
            pltpu.VMEM((G * T * C,), jnp.float32),
            pltpu.VMEM((G * Q * C,), jnp.float32),
        ],
    )
    def body(times_hbm, tq_hbm, obs_hbm, out_hbm,
             times_v, tq_v, lv, rv, wv, in_v, out_v):
        wid = lax.axis_index("s") * _NUM_CORES + lax.axis_index("c")

        pltpu.sync_copy(times_hbm, times_v)
        pltpu.sync_copy(tq_hbm, tq_v)

        for qb in range(qb_n):
            tq = tq_v[pl.ds(qb * _L, _L)]

            cnt = jnp.zeros((_L,), jnp.int32)
            one = jnp.ones((_L,), jnp.int32)
            zero = jnp.zeros((_L,), jnp.int32)
            for tb in range(T // _L):
                tvec = times_v[pl.ds(tb * _L, _L)]
                for lane in range(_L):
                    cnt = cnt + jnp.where(tvec[lane] < tq, one, zero)
            right = jnp.minimum(cnt, T - 1)
            left_pre = jnp.maximum(right - 1, 0)
            left_m1 = left_pre - 1
            zf = jnp.zeros((_L,), jnp.float32)
            tl_pre, tl_m1, t_right = zf, zf, zf
            for tb in range(T // _L):
                tvec = times_v[pl.ds(tb * _L, _L)]
                for lane in range(_L):
                    ts = tvec[lane]
                    ti = tb * _L + lane
                    tl_pre = jnp.where(left_pre == ti, ts, tl_pre)
                    tl_m1 = jnp.where(left_m1 == ti, ts, tl_m1)
                    t_right = jnp.where(right == ti, ts, t_right)
            on_grid = (tl_pre == tq) & (left_pre > 0)
            left = jnp.where(on_grid, left_pre - 1, left_pre)
            t_left = jnp.where(on_grid, tl_m1, tl_pre)
            td = t_right - t_left
            td = jnp.where(td == 0.0, jnp.float32(1e-6), td)
            w = (tq - t_left) / td
            lv[pl.ds(qb * _L, _L)] = left * C
            rv[pl.ds(qb * _L, _L)] = right * C
            wv[pl.ds(qb * _L, _L)] = w

        e0_base = wid * e_per_w

        def group_body(gi, _):
            e0 = e0_base + gi * G
            pltpu.sync_copy(obs_hbm.at[pl.ds(e0 * in_w, G * in_w)], in_v)

            def entity_body(e, _):
                base_in = e * in_w
                base_out = e * out_w

                def qb_body(qb, _):
                    lvec = lv[pl.ds(qb * _L, _L)]
                    rvec = rv[pl.ds(qb * _L, _L)]
                    wvec = wv[pl.ds(qb * _L, _L)]
                    o0 = base_out + qb * _L * C
                    for lane in range(_L):
                        l32 = base_in + lvec[lane]
                        r32 = base_in + rvec[lane]
                        w = wvec[lane]
                        o = o0 + lane * C
                        for ch in range(ch_n):
                            xl = in_v[pl.ds(l32 + ch * _L, _L)]
                            xr = in_v[pl.ds(r32 + ch * _L, _L)]
                            out_v[pl.ds(o + ch * _L, _L)] = xl + w * (xr - xl)
                    return 0

                lax.fori_loop(0, qb_n, qb_body, 0)
                return 0

            lax.fori_loop(0, G, entity_body, 0)
            pltpu.sync_copy(out_v, out_hbm.at[pl.ds(e0 * out_w, G * out_w)])
            return 0

        lax.fori_loop(0, ng, group_body, 0)

    return body


def kernel(times, observations, t_query):
    B1, B2, T, C = observations.shape
    Q = t_query.shape[0]
    N = B1 * B2
    obs_flat = observations.reshape(N * T * C)
    fn = _make_kernel(N, T, C, Q, G=8)
    out_flat = fn(times, t_query, obs_flat)
    return out_flat.reshape(B1, B2, Q, C)


# speedup vs baseline: 1.0120x; 1.0120x over previous
import functools

import jax
import jax.numpy as jnp
from jax import lax
from jax.experimental import pallas as pl
from jax.experimental.pallas import tpu as pltpu
from jax.experimental.pallas import tpu_sc as plsc

_NUM_CORES = 2
_NUM_SUBCORES = 16
_NW = _NUM_CORES * _NUM_SUBCORES
_L = 16


def _make_kernel(N, T, C, Q, G):
    assert N % (_NW * G) == 0 and C % _L == 0 and T % _L == 0
    e_per_w = N // _NW
    ng = e_per_w // G
    in_w = T * C
    out_w = Q * C
    qb_n = Q // _L
    ch_n = C // _L

    mesh = plsc.VectorSubcoreMesh(core_axis_name="c", subcore_axis_name="s")

    @functools.partial(
        pl.kernel,
        out_type=jax.ShapeDtypeStruct((N * out_w,), jnp.float32),
        mesh=mesh,
        scratch_types=[
            pltpu.VMEM((T,), jnp.float32),
            pltpu.VMEM((Q,), jnp.float32),
            pltpu.VMEM((G * T * C,), jnp.float32),
            pltpu.VMEM((G * Q * C,), jnp.float32),
        ],
    )
    def body(times_hbm, tq_hbm, obs_hbm, out_hbm, times_v, tq_v, in_v, out_v):
        wid = lax.axis_index("s") * _NUM_CORES + lax.axis_index("c")

        pltpu.sync_copy(times_hbm, times_v)
        pltpu.sync_copy(tq_hbm, tq_v)

        l_offs = []
        r_offs = []
        ws = []
        for qb in range(qb_n):
            tq = tq_v[pl.ds(qb * _L, _L)]

            def count_tb(tb, cnt):
                tvec = times_v[pl.ds(tb * _L, _L)]
                one = jnp.ones((_L,), jnp.int32)
                zero = jnp.zeros((_L,), jnp.int32)
                for lane in range(_L):
                    cnt = cnt + jnp.where(tvec[lane] < tq, one, zero)
                return cnt

            cnt = lax.fori_loop(0, T // _L, count_tb,
                                jnp.zeros((_L,), jnp.int32))
            right = jnp.minimum(cnt, T - 1)
            left_pre = jnp.maximum(right - 1, 0)
            left_m1 = left_pre - 1

            def sel_tb(tb, carry):
                tl_pre, tl_m1, t_right = carry
                tvec = times_v[pl.ds(tb * _L, _L)]
                for lane in range(_L):
                    ts = tvec[lane]
                    ti = tb * _L + lane
                    tl_pre = jnp.where(left_pre == ti, ts, tl_pre)
                    tl_m1 = jnp.where(left_m1 == ti, ts, tl_m1)
                    t_right = jnp.where(right == ti, ts, t_right)
                return tl_pre, tl_m1, t_right

            zf = jnp.zeros((_L,), jnp.float32)
            tl_pre, tl_m1, t_right = lax.fori_loop(
                0, T // _L, sel_tb, (zf, zf, zf))

            on_grid = (tl_pre == tq) & (left_pre > 0)
            left = jnp.where(on_grid, left_pre - 1, left_pre)
            t_left = jnp.where(on_grid, tl_m1, tl_pre)
            td = t_right - t_left
            td = jnp.where(td == 0.0, jnp.float32(1e-6), td)
            w = (tq - t_left) / td
            lC = left * C
            rC = right * C
            for lane in range(_L):
                l_offs.append(lC[lane])
                r_offs.append(rC[lane])
                ws.append(w[lane])

        e0_base = wid * e_per_w

        def group_body(gi, _):
            e0 = e0_base + gi * G
            pltpu.sync_copy(obs_hbm.at[pl.ds(e0 * in_w, G * in_w)], in_v)

            def entity_body(e, _):
                base_in = e * in_w
                base_out = e * out_w
                for q in range(Q):
                    lq = base_in + l_offs[q]
                    rq = base_in + r_offs[q]
                    wq = ws[q]
                    o = base_out + q * C
                    for ch in range(ch_n):
                        xl = in_v[pl.ds(lq + ch * _L, _L)]
                        xr = in_v[pl.ds(rq + ch * _L, _L)]
                        out_v[pl.ds(o + ch * _L, _L)] = xl + wq * (xr - xl)
                return 0

            lax.fori_loop(0, G, entity_body, 0)
            pltpu.sync_copy(out_v, out_hbm.at[pl.ds(e0 * out_w, G * out_w)])
            return 0

        lax.fori_loop(0, ng, group_body, 0)

    return body


def kernel(times, observations, t_query):
    B1, B2, T, C = observations.shape
    Q = t_query.shape[0]
    N = B1 * B2
    obs_flat = observations.reshape(N * T * C)
    fn = _make_kernel(N, T, C, Q, G=8)
    out_flat = fn(times, t_query, obs_flat)
    return out_flat.reshape(B1, B2, Q, C)


# speedup vs baseline: 1.1104x; 1.0972x over previous
import functools

import jax
import jax.numpy as jnp
from jax import lax
from jax.experimental import pallas as pl
from jax.experimental.pallas import tpu as pltpu
from jax.experimental.pallas import tpu_sc as plsc

_NUM_CORES = 2
_NUM_SUBCORES = 16
_NW = _NUM_CORES * _NUM_SUBCORES
_L = 16


def _make_kernel(N, T, C, Q, G):
    assert N % (_NW * 2 * G) == 0 and C % _L == 0 and T % _L == 0
    e_per_w = N // _NW
    ng = e_per_w // G
    in_w = T * C
    out_w = Q * C
    qb_n = Q // _L
    ch_n = C // _L

    mesh = plsc.VectorSubcoreMesh(core_axis_name="c", subcore_axis_name="s")

    @functools.partial(
        pl.kernel,
        out_type=jax.ShapeDtypeStruct((N, out_w), jnp.float32),
        mesh=mesh,
        scratch_types=[
            pltpu.VMEM((T,), jnp.float32),
            pltpu.VMEM((Q,), jnp.float32),
            pltpu.VMEM((2, G, in_w), jnp.float32),
            pltpu.VMEM((2, G, out_w), jnp.float32),
            pltpu.SemaphoreType.DMA,
            pltpu.SemaphoreType.DMA,
            pltpu.SemaphoreType.DMA,
            pltpu.SemaphoreType.DMA,
        ],
    )
    def body(times_hbm, tq_hbm, obs_hbm, out_hbm,
             times_v, tq_v, in_v, out_v, si0, si1, so0, so1):
        wid = lax.axis_index("s") * _NUM_CORES + lax.axis_index("c")
        sin = (si0, si1)
        sout = (so0, so1)

        pltpu.sync_copy(times_hbm, times_v)
        pltpu.sync_copy(tq_hbm, tq_v)

        l_offs = []
        r_offs = []
        ws = []
        for qb in range(qb_n):
            tq = tq_v[pl.ds(qb * _L, _L)]

            def count_tb(tb, cnt):
                tvec = times_v[pl.ds(tb * _L, _L)]
                one = jnp.ones((_L,), jnp.int32)
                zero = jnp.zeros((_L,), jnp.int32)
                for lane in range(_L):
                    cnt = cnt + jnp.where(tvec[lane] < tq, one, zero)
                return cnt

            cnt = lax.fori_loop(0, T // _L, count_tb,
                                jnp.zeros((_L,), jnp.int32))
            right = jnp.minimum(cnt, T - 1)
            left_pre = jnp.maximum(right - 1, 0)
            left_m1 = left_pre - 1

            def sel_tb(tb, carry):
                tl_pre, tl_m1, t_right = carry
                tvec = times_v[pl.ds(tb * _L, _L)]
                for lane in range(_L):
                    ts = tvec[lane]
                    ti = tb * _L + lane
                    tl_pre = jnp.where(left_pre == ti, ts, tl_pre)
                    tl_m1 = jnp.where(left_m1 == ti, ts, tl_m1)
                    t_right = jnp.where(right == ti, ts, t_right)
                return tl_pre, tl_m1, t_right

            zf = jnp.zeros((_L,), jnp.float32)
            tl_pre, tl_m1, t_right = lax.fori_loop(
                0, T // _L, sel_tb, (zf, zf, zf))

            on_grid = (tl_pre == tq) & (left_pre > 0)
            left = jnp.where(on_grid, left_pre - 1, left_pre)
            t_left = jnp.where(on_grid, tl_m1, tl_pre)
            td = t_right - t_left
            td = jnp.where(td == 0.0, jnp.float32(1e-6), td)
            w = (tq - t_left) / td
            lC = left * C
            rC = right * C
            for lane in range(_L):
                l_offs.append(lC[lane])
                r_offs.append(rC[lane])
                ws.append(w[lane])

        e0_base = wid * e_per_w

        def start_in(gi, slot):
            e0 = e0_base + gi * G
            return pltpu.async_copy(
                obs_hbm.at[pl.ds(e0, G)], in_v.at[slot], sin[slot])

        def start_out(gi, slot):
            e0 = e0_base + gi * G
            return pltpu.async_copy(
                out_v.at[slot], out_hbm.at[pl.ds(e0, G)], sout[slot])

        def compute_group(slot):
            def entity_body(e, _):
                base_out = e * out_w
                for q in range(Q):
                    lq = pl.multiple_of(l_offs[q], _L)
                    rq = pl.multiple_of(r_offs[q], _L)
                    wq = ws[q]
                    o = base_out + q * C
                    for ch in range(ch_n):
                        xl = in_v[slot, e, pl.ds(lq + ch * _L, _L)]
                        xr = in_v[slot, e, pl.ds(rq + ch * _L, _L)]
                        out_v[slot, e, pl.ds(o - base_out + ch * _L, _L)] = (
                            xl + wq * (xr - xl))
                return 0

            lax.fori_loop(0, G, entity_body, 0)

        start_in(0, 0).wait()

        def pair_body(p, _):
            gi = p * 2
            for b in range(2):
                g = gi + b
                nxt = g + 1

                @pl.when(nxt < ng)
                def _():
                    start_in(nxt, 1 - b)

                @pl.when(g >= 2)
                def _():
                    pltpu.make_async_copy(
                        out_v.at[b], out_hbm.at[pl.ds(0, G)], sout[b]).wait()

                compute_group(b)
                start_out(g, b)

                @pl.when(nxt < ng)
                def _():
                    pltpu.make_async_copy(
                        obs_hbm.at[pl.ds(0, G)], in_v.at[1 - b],
                        sin[1 - b]).wait()
            return 0

        lax.fori_loop(0, ng // 2, pair_body, 0)
        pltpu.make_async_copy(
            out_v.at[0], out_hbm.at[pl.ds(0, G)], sout[0]).wait()
        pltpu.make_async_copy(
            out_v.at[1], out_hbm.at[pl.ds(0, G)], sout[1]).wait()

    return body


def kernel(times, observations, t_query):
    B1, B2, T, C = observations.shape
    Q = t_query.shape[0]
    N = B1 * B2
    obs_2d = observations.reshape(N, T * C)
    fn = _make_kernel(N, T, C, Q, G=8)
    out_2d = fn(times, t_query, obs_2d)
    return out_2d.reshape(B1, B2, Q, C)
